# Initial kernel scaffold; baseline (speedup 1.0000x reference)
import functools

import jax
import jax.numpy as jnp
from jax import lax
from jax.experimental import pallas as pl
from jax.experimental.pallas import tpu as pltpu
from jax.experimental.pallas import tpu_sc as plsc

_N = 10000
_F = 128
_C = 4
_B = 128
_KB = 16
_NBLK = 10
_NSUB = 16
_EPAD = _NSUB * _NBLK * _KB * _B
_NTBL = _N + 16
_RPS = _N // _NSUB
_NPAD = 10240

_HIGH = lax.Precision.HIGHEST


def _sc_degree(edges3):
    mesh = plsc.VectorSubcoreMesh(core_axis_name="c", subcore_axis_name="s")
    nbd = _EPAD // (32 * _B)
    rz = _NPAD // _NSUB

    @functools.partial(
        pl.kernel,
        out_type=jax.ShapeDtypeStruct((2, _NPAD), jnp.float32),
        mesh=mesh,
        scratch_types=[
            pltpu.VMEM((8, _B), jnp.int32),
            pltpu.VMEM((_B,), jnp.float32),
            pltpu.VMEM((rz,), jnp.float32),
            pltpu.VMEM_SHARED((_NPAD,), jnp.float32),
        ],
    )
    def deg_kernel(e_hbm, o_hbm, dbuf, ones, zbuf, table):
        core = lax.axis_index("c")
        sub = lax.axis_index("s")
        wid = core * _NSUB + sub
        for q in range(_B // 16):
            ones[pl.ds(q * 16, 16)] = jnp.ones((16,), jnp.float32)
        for q in range(rz // 16):
            zbuf[pl.ds(q * 16, 16)] = jnp.zeros((16,), jnp.float32)
        pltpu.sync_copy(zbuf, table.at[pl.ds(sub * rz, rz)])
        plsc.subcore_barrier()

        @pl.loop(0, nbd // 8)
        def _(i):
            pltpu.sync_copy(e_hbm.at[1, pl.ds(wid * nbd + i * 8, 8), :], dbuf)
            for k in range(8):
                pltpu.sync_copy(ones, table.at[dbuf.at[k]], add=True)

        plsc.subcore_barrier()
        pltpu.sync_copy(table.at[pl.ds(sub * rz, rz)],
                        o_hbm.at[core, pl.ds(sub * rz, rz)])

    return deg_kernel(edges3)


def _sc_aggregate(hs_flat, edges3):
    mesh = plsc.VectorSubcoreMesh(core_axis_name="c", subcore_axis_name="s")
    nb = _NBLK * _KB

    @functools.partial(
        pl.kernel,
        out_type=jax.ShapeDtypeStruct((_C * _N, _F), jnp.float32),
        mesh=mesh,
        scratch_types=[
            pltpu.VMEM((2, 2, _KB, _B), jnp.int32),
            pltpu.VMEM((2, _B), jnp.int32),
            pltpu.VMEM((2, _B, _F), jnp.float32),
            pltpu.VMEM_SHARED((_NTBL, _F), jnp.float32),
            pltpu.SemaphoreType.DMA,
            pltpu.SemaphoreType.DMA,
            pltpu.SemaphoreType.DMA,
            pltpu.SemaphoreType.DMA,
        ],
    )
    def agg_kernel(hs_hbm, e_hbm, o_hbm, ibuf, sbuf, rbuf, table,
                   si0, si1, sg0, sg1):
        core = lax.axis_index("c")
        sub = lax.axis_index("s")
        sem_i = [si0, si1]
        sem_g = [sg0, sg1]

        def idx_dma(blk, slot):
            return pltpu.make_async_copy(
                e_hbm.at[:, pl.ds(sub * nb + blk * _KB, _KB), :],
                ibuf.at[slot], sem_i[slot])

        def gather_dma(slot):
            return pltpu.make_async_copy(
                hs_hbm.at[sbuf.at[slot]], rbuf.at[slot], sem_g[slot])

        def comp_srcoff(islot, j, gslot, row0):
            for q in range(_B // 16):
                sl = pl.ds(q * 16, 16)
                sbuf[gslot, sl] = ibuf[islot, 0, j, sl] + row0

        for p in range(2):
            cid = core * 2 + p
            row0 = cid * _N
            pltpu.sync_copy(hs_hbm.at[pl.ds(row0 + sub * _RPS, _RPS)],
                            table.at[pl.ds(sub * _RPS, _RPS)])
            plsc.subcore_barrier()

            idx_dma(0, 0).start()
            idx_dma(0, 0).wait()
            comp_srcoff(0, 0, 0, row0)
            gather_dma(0).start()

            @pl.loop(0, _NBLK, step=2)
            def _(b):
                for half in range(2):
                    blk = b + half
                    idx_dma(lax.rem(blk + 1, _NBLK), 1 - half).start()
                    for j in range(_KB):
                        cur = j % 2
                        nxt = 1 - cur
                        gather_dma(cur).wait()
                        if j < _KB - 1:
                            comp_srcoff(half, j + 1, nxt, row0)
                        else:
                            idx_dma(0, 1 - half).wait()
                            comp_srcoff(1 - half, 0, nxt, row0)
                        gather_dma(nxt).start()
                        pltpu.sync_copy(rbuf.at[cur],
                                        table.at[ibuf.at[half, 1, j]],
                                        add=True)

            gather_dma(0).wait()
            plsc.subcore_barrier()
            pltpu.sync_copy(table.at[pl.ds(sub * _RPS, _RPS)],
                            o_hbm.at[pl.ds(row0 + sub * _RPS, _RPS)])
            plsc.subcore_barrier()

    return agg_kernel(hs_flat, edges3)


_R = 1000


def _tc_first(x_cat, degp3, W0):
    def body(x_ref, d_ref, w_ref, o_ref, dinv_ref):
        deg = d_ref[0] + d_ref[1] + 1.0
        dinv = lax.rsqrt(deg)
        dinv_ref[...] = dinv
        for k in range(2):
            h = lax.dot_general(x_ref[k], w_ref[...],
                                (((1,), (0,)), ((), ())), precision=_HIGH)
            hs = h * dinv
            o_ref[2 * k] = hs[:, :_F]
            o_ref[2 * k + 1] = hs[:, _F:]

    g = _N // _R
    return pl.pallas_call(
        body,
        grid=(g,),
        in_specs=[
            pl.BlockSpec((2, _R, _F), lambda i: (0, i, 0)),
            pl.BlockSpec((2, _R, 1), lambda i: (0, i, 0)),
            pl.BlockSpec((_F, 2 * _F), lambda i: (0, 0)),
        ],
        out_specs=[
            pl.BlockSpec((_C, _R, _F), lambda i: (0, i, 0)),
            pl.BlockSpec((_R, 1), lambda i: (i, 0)),
        ],
        out_shape=[
            jax.ShapeDtypeStruct((_C, _N, _F), jnp.float32),
            jax.ShapeDtypeStruct((_N, 1), jnp.float32),
        ],
    )(x_cat, degp3, W0)


def _tc_mid(agg1, dinv, W1, b0):
    def body(a_ref, d_ref, w_ref, b_ref, o_ref):
        dinv = d_ref[...]
        for k in range(2):
            a = jnp.concatenate([a_ref[2 * k], a_ref[2 * k + 1]], axis=1)
            a = a * dinv + b_ref[...]
            h = lax.dot_general(a, w_ref[...],
                                (((1,), (0,)), ((), ())), precision=_HIGH)
            hs = h * dinv
            o_ref[2 * k] = hs[:, :_F]
            o_ref[2 * k + 1] = hs[:, _F:]

    g = _N // _R
    return pl.pallas_call(
        body,
        grid=(g,),
        in_specs=[
            pl.BlockSpec((_C, _R, _F), lambda i: (0, i, 0)),
            pl.BlockSpec((_R, 1), lambda i: (i, 0)),
            pl.BlockSpec((2 * _F, 2 * _F), lambda i: (0, 0)),
            pl.BlockSpec((1, 2 * _F), lambda i: (0, 0)),
        ],
        out_specs=pl.BlockSpec((_C, _R, _F), lambda i: (0, i, 0)),
        out_shape=jax.ShapeDtypeStruct((_C, _N, _F), jnp.float32),
    )(agg1, dinv, W1, b0)


def _tc_final(agg2, dinv, b1, x):
    def body(a_ref, d_ref, b_ref, x_ref, z_ref, z1_ref, z2_ref):
        dinv = d_ref[...]
        xv = x_ref[...]
        z1 = jnp.concatenate([a_ref[0], a_ref[1]], axis=1) * dinv + b_ref[...]
        z2 = jnp.concatenate([a_ref[2], a_ref[3]], axis=1) * dinv + b_ref[...]
        ss = (jnp.sum(z1 * z1, axis=1, keepdims=True)
              + jnp.sum(z2 * z2, axis=1, keepdims=True)
              + jnp.sum(xv * xv, axis=1, keepdims=True))
        rn = lax.rsqrt(ss)
        z1_ref[...] = z1
        z2_ref[...] = z2
        z_ref[:, 0:2 * _F] = z1 * rn
        z_ref[:, 2 * _F:4 * _F] = z2 * rn
        z_ref[:, 4 * _F:] = xv * rn

    g = _N // _R
    return pl.pallas_call(
        body,
        grid=(g,),
        in_specs=[
            pl.BlockSpec((_C, _R, _F), lambda i: (0, i, 0)),
            pl.BlockSpec((_R, 1), lambda i: (i, 0)),
            pl.BlockSpec((1, 2 * _F), lambda i: (0, 0)),
            pl.BlockSpec((_R, _F), lambda i: (i, 0)),
        ],
        out_specs=[
            pl.BlockSpec((_R, 5 * _F), lambda i: (i, 0)),
            pl.BlockSpec((_R, 2 * _F), lambda i: (i, 0)),
            pl.BlockSpec((_R, 2 * _F), lambda i: (i, 0)),
        ],
        out_shape=[
            jax.ShapeDtypeStruct((_N, 5 * _F), jnp.float32),
            jax.ShapeDtypeStruct((_N, 2 * _F), jnp.float32),
            jax.ShapeDtypeStruct((_N, 2 * _F), jnp.float32),
        ],
    )(agg2, dinv, b1, x)


def kernel(edge_index, x_feature, llmfeatures, W0, b0, W1, b1,
           Wp, bp, Wpred, bpred, Wmlp, bmlp):
    src = edge_index[0]
    dst = edge_index[1]
    pad = _EPAD - src.shape[0]
    src_p = jnp.concatenate([src, jnp.zeros((pad,), jnp.int32)])
    dst_p = jnp.concatenate([dst, jnp.full((pad,), _N, jnp.int32)])
    edges3 = jnp.stack([src_p, dst_p]).reshape(2, _EPAD // _B, _B)

    degp = _sc_degree(edges3)
    degp3 = degp[:, :_N].reshape(2, _N, 1)

    x_cat = jnp.stack([x_feature, llmfeatures])
    hs1, dinv = _tc_first(x_cat, degp3, W0)

    agg1 = _sc_aggregate(hs1.reshape(_C * _N, _F), edges3)
    hs2 = _tc_mid(agg1.reshape(_C, _N, _F), dinv, W1, b0.reshape(1, 2 * _F))

    agg2 = _sc_aggregate(hs2.reshape(_C * _N, _F), edges3)
    z, z1, z2 = _tc_final(agg2.reshape(_C, _N, _F), dinv,
                          b1.reshape(1, 2 * _F), x_feature)
    return (x_feature, z, z1, z2)


# speedup vs baseline: 7.2062x; 7.2062x over previous
import functools

import jax
import jax.numpy as jnp
from jax import lax
from jax.experimental import pallas as pl
from jax.experimental.pallas import tpu as pltpu
from jax.experimental.pallas import tpu_sc as plsc

_N = 10000
_F = 128
_C = 4
_B = 128
_KB = 16
_NBLK = 10
_NSUB = 16
_EPAD = _NSUB * _NBLK * _KB * _B
_NP = 10112
_RPS = _NP // _NSUB
_NPAD = 10240

_HIGH = lax.Precision.HIGHEST


def _sc_degree(edges3):
    mesh = plsc.VectorSubcoreMesh(core_axis_name="c", subcore_axis_name="s", num_cores=2, num_subcores=_NSUB)
    nbd = _EPAD // (32 * _B)
    rz = _NPAD // _NSUB

    @functools.partial(
        pl.kernel,
        out_type=jax.ShapeDtypeStruct((2, _NPAD), jnp.float32),
        mesh=mesh,
        scratch_types=[
            pltpu.VMEM((8, _B), jnp.int32),
            pltpu.VMEM((_B,), jnp.float32),
            pltpu.VMEM((rz,), jnp.float32),
            pltpu.VMEM_SHARED((_NPAD,), jnp.float32),
        ],
    )
    def deg_kernel(e_hbm, o_hbm, dbuf, ones, zbuf, table):
        core = lax.axis_index("c")
        sub = lax.axis_index("s")
        wid = core * _NSUB + sub
        for q in range(_B // 16):
            ones[pl.ds(q * 16, 16)] = jnp.ones((16,), jnp.float32)
        for q in range(rz // 16):
            zbuf[pl.ds(q * 16, 16)] = jnp.zeros((16,), jnp.float32)
        pltpu.sync_copy(zbuf, table.at[pl.ds(sub * rz, rz)])
        plsc.subcore_barrier()

        @pl.loop(0, nbd // 8)
        def _(i):
            pltpu.sync_copy(e_hbm.at[1, pl.ds(wid * nbd + i * 8, 8), :], dbuf)
            for k in range(8):
                pltpu.sync_copy(ones, table.at[dbuf.at[k]], add=True)

        plsc.subcore_barrier()
        pltpu.sync_copy(table.at[pl.ds(sub * rz, rz)],
                        o_hbm.at[core, pl.ds(sub * rz, rz)])

    return deg_kernel(edges3)


def _sc_aggregate(hs_flat, edges3):
    mesh = plsc.VectorSubcoreMesh(core_axis_name="c", subcore_axis_name="s", num_cores=2, num_subcores=_NSUB)
    nb = _NBLK * _KB

    @functools.partial(
        pl.kernel,
        out_type=jax.ShapeDtypeStruct((_C * _NP, _F), jnp.float32),
        mesh=mesh,
        scratch_types=[
            pltpu.VMEM((2, 2, _KB, _B), jnp.int32),
            pltpu.VMEM((2, _B), jnp.int32),
            pltpu.VMEM((2, _B, _F), jnp.float32),
            pltpu.VMEM_SHARED((_NP, _F), jnp.float32),
            pltpu.SemaphoreType.DMA,
            pltpu.SemaphoreType.DMA,
            pltpu.SemaphoreType.DMA,
            pltpu.SemaphoreType.DMA,
        ],
    )
    def agg_kernel(hs_hbm, e_hbm, o_hbm, ibuf, sbuf, rbuf, table,
                   si0, si1, sg0, sg1):
        core = lax.axis_index("c")
        sub = lax.axis_index("s")
        sem_i = [si0, si1]
        sem_g = [sg0, sg1]

        def idx_dma(blk, slot):
            return pltpu.make_async_copy(
                e_hbm.at[:, pl.ds(sub * nb + blk * _KB, _KB), :],
                ibuf.at[slot], sem_i[slot])

        def gather_dma(slot):
            return pltpu.make_async_copy(
                hs_hbm.at[sbuf.at[slot]], rbuf.at[slot], sem_g[slot])

        def comp_srcoff(islot, j, gslot, row0):
            for q in range(_B // 16):
                sl = pl.ds(q * 16, 16)
                sbuf[gslot, sl] = ibuf[islot, 0, j, sl] + row0

        for p in range(2):
            cid = core * 2 + p
            row0 = cid * _NP
            pltpu.sync_copy(hs_hbm.at[pl.ds(row0 + sub * _RPS, _RPS)],
                            table.at[pl.ds(sub * _RPS, _RPS)])
            plsc.subcore_barrier()

            idx_dma(0, 0).start()
            idx_dma(0, 0).wait()
            comp_srcoff(0, 0, 0, row0)
            gather_dma(0).start()

            @pl.loop(0, _NBLK, step=2)
            def _(b):
                for half in range(2):
                    blk = b + half
                    idx_dma(lax.rem(blk + 1, _NBLK), 1 - half).start()
                    for j in range(_KB):
                        cur = j % 2
                        nxt = 1 - cur
                        gather_dma(cur).wait()
                        if j < _KB - 1:
                            comp_srcoff(half, j + 1, nxt, row0)
                        else:
                            idx_dma(0, 1 - half).wait()
                            comp_srcoff(1 - half, 0, nxt, row0)
                        gather_dma(nxt).start()
                        pltpu.sync_copy(rbuf.at[cur],
                                        table.at[ibuf.at[half, 1, j]],
                                        add=True)

            gather_dma(0).wait()
            plsc.subcore_barrier()
            pltpu.sync_copy(table.at[pl.ds(sub * _RPS, _RPS)],
                            o_hbm.at[pl.ds(row0 + sub * _RPS, _RPS)])
            plsc.subcore_barrier()

    return agg_kernel(hs_flat, edges3)


_R = 1000


def _tc_first(x_cat, degp3, W0):
    def body(x_ref, d_ref, w_ref, o_ref, dinv_ref):
        deg = d_ref[0] + d_ref[1] + 1.0
        dinv = lax.rsqrt(deg)
        dinv_ref[...] = dinv
        for k in range(2):
            h = lax.dot_general(x_ref[k], w_ref[...],
                                (((1,), (0,)), ((), ())), precision=_HIGH)
            hs = h * dinv
            o_ref[2 * k] = hs[:, :_F]
            o_ref[2 * k + 1] = hs[:, _F:]

    g = _N // _R
    return pl.pallas_call(
        body,
        grid=(g,),
        in_specs=[
            pl.BlockSpec((2, _R, _F), lambda i: (0, i, 0)),
            pl.BlockSpec((2, _R, 1), lambda i: (0, i, 0)),
            pl.BlockSpec((_F, 2 * _F), lambda i: (0, 0)),
        ],
        out_specs=[
            pl.BlockSpec((_C, _R, _F), lambda i: (0, i, 0)),
            pl.BlockSpec((_R, 1), lambda i: (i, 0)),
        ],
        out_shape=[
            jax.ShapeDtypeStruct((_C, _NP, _F), jnp.float32),
            jax.ShapeDtypeStruct((_N, 1), jnp.float32),
        ],
    )(x_cat, degp3, W0)


def _tc_mid(agg1, dinv, W1, b0):
    def body(a_ref, d_ref, w_ref, b_ref, o_ref):
        dinv = d_ref[...]
        for k in range(2):
            a = jnp.concatenate([a_ref[2 * k], a_ref[2 * k + 1]], axis=1)
            a = a * dinv + b_ref[...]
            h = lax.dot_general(a, w_ref[...],
                                (((1,), (0,)), ((), ())), precision=_HIGH)
            hs = h * dinv
            o_ref[2 * k] = hs[:, :_F]
            o_ref[2 * k + 1] = hs[:, _F:]

    g = _N // _R
    return pl.pallas_call(
        body,
        grid=(g,),
        in_specs=[
            pl.BlockSpec((_C, _R, _F), lambda i: (0, i, 0)),
            pl.BlockSpec((_R, 1), lambda i: (i, 0)),
            pl.BlockSpec((2 * _F, 2 * _F), lambda i: (0, 0)),
            pl.BlockSpec((1, 2 * _F), lambda i: (0, 0)),
        ],
        out_specs=pl.BlockSpec((_C, _R, _F), lambda i: (0, i, 0)),
        out_shape=jax.ShapeDtypeStruct((_C, _NP, _F), jnp.float32),
    )(agg1, dinv, W1, b0)


def _tc_final(agg2, dinv, b1, x):
    def body(a_ref, d_ref, b_ref, x_ref, z_ref, z1_ref, z2_ref):
        dinv = d_ref[...]
        xv = x_ref[...]
        z1 = jnp.concatenate([a_ref[0], a_ref[1]], axis=1) * dinv + b_ref[...]
        z2 = jnp.concatenate([a_ref[2], a_ref[3]], axis=1) * dinv + b_ref[...]
        ss = (jnp.sum(z1 * z1, axis=1, keepdims=True)
              + jnp.sum(z2 * z2, axis=1, keepdims=True)
              + jnp.sum(xv * xv, axis=1, keepdims=True))
        rn = lax.rsqrt(ss)
        z1_ref[...] = z1
        z2_ref[...] = z2
        z_ref[:, 0:2 * _F] = z1 * rn
        z_ref[:, 2 * _F:4 * _F] = z2 * rn
        z_ref[:, 4 * _F:] = xv * rn

    g = _N // _R
    return pl.pallas_call(
        body,
        grid=(g,),
        in_specs=[
            pl.BlockSpec((_C, _R, _F), lambda i: (0, i, 0)),
            pl.BlockSpec((_R, 1), lambda i: (i, 0)),
            pl.BlockSpec((1, 2 * _F), lambda i: (0, 0)),
            pl.BlockSpec((_R, _F), lambda i: (i, 0)),
        ],
        out_specs=[
            pl.BlockSpec((_R, 5 * _F), lambda i: (i, 0)),
            pl.BlockSpec((_R, 2 * _F), lambda i: (i, 0)),
            pl.BlockSpec((_R, 2 * _F), lambda i: (i, 0)),
        ],
        out_shape=[
            jax.ShapeDtypeStruct((_N, 5 * _F), jnp.float32),
            jax.ShapeDtypeStruct((_N, 2 * _F), jnp.float32),
            jax.ShapeDtypeStruct((_N, 2 * _F), jnp.float32),
        ],
    )(agg2, dinv, b1, x)


def kernel(edge_index, x_feature, llmfeatures, W0, b0, W1, b1,
           Wp, bp, Wpred, bpred, Wmlp, bmlp):
    src = edge_index[0]
    dst = edge_index[1]
    pad = _EPAD - src.shape[0]
    src_p = jnp.concatenate([src, jnp.zeros((pad,), jnp.int32)])
    dst_p = jnp.concatenate([dst, jnp.full((pad,), _N, jnp.int32)])
    edges3 = jnp.stack([src_p, dst_p]).reshape(2, _EPAD // _B, _B)

    degp = _sc_degree(edges3)
    degp3 = degp[:, :_N].reshape(2, _N, 1)

    x_cat = jnp.stack([x_feature, llmfeatures])
    hs1, dinv = _tc_first(x_cat, degp3, W0)

    agg1 = _sc_aggregate(hs1.reshape(_C * _NP, _F), edges3)
    hs2 = _tc_mid(agg1.reshape(_C, _NP, _F), dinv, W1, b0.reshape(1, 2 * _F))

    agg2 = _sc_aggregate(hs2.reshape(_C * _NP, _F), edges3)
    z, z1, z2 = _tc_final(agg2.reshape(_C, _NP, _F), dinv,
                          b1.reshape(1, 2 * _F), x_feature)
    return (x_feature, z, z1, z2)


# speedup vs baseline: 7.3655x; 1.0221x over previous
import functools

import jax
import jax.numpy as jnp
from jax import lax
from jax.experimental import pallas as pl
from jax.experimental.pallas import tpu as pltpu
from jax.experimental.pallas import tpu_sc as plsc

_N = 10000
_F = 128
_C = 4
_EROW = 128
_GB = 64
_KB = 8
_NBLK = 20
_NSUB = 16
_EPAD = _NSUB * _NBLK * _KB * _EROW
_NP = 10112
_RPS = _NP // _NSUB
_NPAD = 10240

_HIGH = lax.Precision.HIGHEST


def _sc_degree(edges3):
    mesh = plsc.VectorSubcoreMesh(core_axis_name="c", subcore_axis_name="s", num_cores=2, num_subcores=_NSUB)
    nbd = _EPAD // (32 * _EROW)
    rz = _NPAD // _NSUB

    @functools.partial(
        pl.kernel,
        out_type=jax.ShapeDtypeStruct((2, _NPAD), jnp.float32),
        mesh=mesh,
        scratch_types=[
            pltpu.VMEM((8, _EROW), jnp.int32),
            pltpu.VMEM((_EROW,), jnp.float32),
            pltpu.VMEM((rz,), jnp.float32),
            pltpu.VMEM_SHARED((_NPAD,), jnp.float32),
        ],
    )
    def deg_kernel(e_hbm, o_hbm, dbuf, ones, zbuf, table):
        core = lax.axis_index("c")
        sub = lax.axis_index("s")
        wid = core * _NSUB + sub
        for q in range(_EROW // 16):
            ones[pl.ds(q * 16, 16)] = jnp.ones((16,), jnp.float32)
        for q in range(rz // 16):
            zbuf[pl.ds(q * 16, 16)] = jnp.zeros((16,), jnp.float32)
        pltpu.sync_copy(zbuf, table.at[pl.ds(sub * rz, rz)])
        plsc.subcore_barrier()

        @pl.loop(0, nbd // 8)
        def _(i):
            pltpu.sync_copy(e_hbm.at[1, pl.ds(wid * nbd + i * 8, 8), :], dbuf)
            for k in range(8):
                pltpu.sync_copy(ones, table.at[dbuf.at[k]], add=True)

        plsc.subcore_barrier()
        pltpu.sync_copy(table.at[pl.ds(sub * rz, rz)],
                        o_hbm.at[core, pl.ds(sub * rz, rz)])

    return deg_kernel(edges3)


def _sc_aggregate(hs_flat, edges3):
    mesh = plsc.VectorSubcoreMesh(core_axis_name="c", subcore_axis_name="s", num_cores=2, num_subcores=_NSUB)
    rows_ps = _NBLK * _KB

    @functools.partial(
        pl.kernel,
        out_type=jax.ShapeDtypeStruct((_C * _NP, _F), jnp.float32),
        mesh=mesh,
        scratch_types=[
            pltpu.VMEM((2, 2, _KB, _EROW), jnp.int32),
            pltpu.VMEM((4, _GB), jnp.int32),
            pltpu.VMEM((1, _GB), jnp.int32),
            pltpu.VMEM((4, _GB, _F), jnp.float32),
            pltpu.VMEM_SHARED((_NP, _F), jnp.float32),
            pltpu.SemaphoreType.DMA,
            pltpu.SemaphoreType.DMA,
            pltpu.SemaphoreType.DMA,
            pltpu.SemaphoreType.DMA,
            pltpu.SemaphoreType.DMA,
            pltpu.SemaphoreType.DMA,
        ],
    )
    def agg_kernel(hs_hbm, e_hbm, o_hbm, ibuf, sbuf, dbuf, rbuf, table,
                   si0, si1, sg0, sg1, sg2, sg3):
        core = lax.axis_index("c")
        sub = lax.axis_index("s")
        sem_i = [si0, si1]
        sem_g = [sg0, sg1, sg2, sg3]

        def idx_dma(blk, slot):
            return pltpu.make_async_copy(
                e_hbm.at[:, pl.ds(sub * rows_ps + blk * _KB, _KB), :],
                ibuf.at[slot], sem_i[slot])

        def gather_dma(slot):
            return pltpu.make_async_copy(
                hs_hbm.at[sbuf.at[slot]], rbuf.at[slot], sem_g[slot])

        def comp_srcoff(islot, j, gslot, row0):
            row, off = j // 2, _GB * (j % 2)
            for q in range(_GB // 16):
                sbuf[gslot, pl.ds(q * 16, 16)] = (
                    ibuf[islot, 0, row, pl.ds(off + q * 16, 16)] + row0)

        def scatter_add(islot, j, gslot):
            row, off = j // 2, _GB * (j % 2)
            for q in range(_GB // 16):
                dbuf[0, pl.ds(q * 16, 16)] = (
                    ibuf[islot, 1, row, pl.ds(off + q * 16, 16)])
            pltpu.sync_copy(rbuf.at[gslot], table.at[dbuf.at[0]], add=True)

        nbat = 2 * _KB

        for p in range(2):
            cid = core * 2 + p
            row0 = cid * _NP
            pltpu.sync_copy(hs_hbm.at[pl.ds(row0 + sub * _RPS, _RPS)],
                            table.at[pl.ds(sub * _RPS, _RPS)])
            plsc.subcore_barrier()

            idx_dma(0, 0).start()
            idx_dma(0, 0).wait()
            comp_srcoff(0, 0, 0, row0)
            gather_dma(0).start()
            comp_srcoff(0, 1, 1, row0)
            gather_dma(1).start()

            @pl.loop(0, _NBLK, step=2)
            def _(b):
                for half in range(2):
                    blk = b + half
                    idx_dma(lax.rem(blk + 1, _NBLK), 1 - half).start()
                    for j in range(nbat):
                        cur = j % 4
                        nxt = (j + 2) % 4
                        gather_dma(cur).wait()
                        if j == nbat - 3:
                            idx_dma(0, 1 - half).wait()
                        if j < nbat - 2:
                            comp_srcoff(half, j + 2, nxt, row0)
                        else:
                            comp_srcoff(1 - half, j + 2 - nbat, nxt, row0)
                        gather_dma(nxt).start()
                        scatter_add(half, j, cur)

            gather_dma(0).wait()
            gather_dma(1).wait()
            plsc.subcore_barrier()
            pltpu.sync_copy(table.at[pl.ds(sub * _RPS, _RPS)],
                            o_hbm.at[pl.ds(row0 + sub * _RPS, _RPS)])
            plsc.subcore_barrier()

    return agg_kernel(hs_flat, edges3)


_R = 1000


def _tc_first(x_cat, degp3, W0):
    def body(x_ref, d_ref, w_ref, o_ref, dinv_ref):
        deg = d_ref[0] + d_ref[1] + 1.0
        dinv = lax.rsqrt(deg)
        dinv_ref[...] = dinv
        for k in range(2):
            h = lax.dot_general(x_ref[k], w_ref[...],
                                (((1,), (0,)), ((), ())), precision=_HIGH)
            hs = h * dinv
            o_ref[2 * k] = hs[:, :_F]
            o_ref[2 * k + 1] = hs[:, _F:]

    g = _N // _R
    return pl.pallas_call(
        body,
        grid=(g,),
        in_specs=[
            pl.BlockSpec((2, _R, _F), lambda i: (0, i, 0)),
            pl.BlockSpec((2, _R, 1), lambda i: (0, i, 0)),
            pl.BlockSpec((_F, 2 * _F), lambda i: (0, 0)),
        ],
        out_specs=[
            pl.BlockSpec((_C, _R, _F), lambda i: (0, i, 0)),
            pl.BlockSpec((_R, 1), lambda i: (i, 0)),
        ],
        out_shape=[
            jax.ShapeDtypeStruct((_C, _NP, _F), jnp.float32),
            jax.ShapeDtypeStruct((_N, 1), jnp.float32),
        ],
    )(x_cat, degp3, W0)


def _tc_mid(agg1, dinv, W1, b0):
    def body(a_ref, d_ref, w_ref, b_ref, o_ref):
        dinv = d_ref[...]
        for k in range(2):
            a = jnp.concatenate([a_ref[2 * k], a_ref[2 * k + 1]], axis=1)
            a = a * dinv + b_ref[...]
            h = lax.dot_general(a, w_ref[...],
                                (((1,), (0,)), ((), ())), precision=_HIGH)
            hs = h * dinv
            o_ref[2 * k] = hs[:, :_F]
            o_ref[2 * k + 1] = hs[:, _F:]

    g = _N // _R
    return pl.pallas_call(
        body,
        grid=(g,),
        in_specs=[
            pl.BlockSpec((_C, _R, _F), lambda i: (0, i, 0)),
            pl.BlockSpec((_R, 1), lambda i: (i, 0)),
            pl.BlockSpec((2 * _F, 2 * _F), lambda i: (0, 0)),
            pl.BlockSpec((1, 2 * _F), lambda i: (0, 0)),
        ],
        out_specs=pl.BlockSpec((_C, _R, _F), lambda i: (0, i, 0)),
        out_shape=jax.ShapeDtypeStruct((_C, _NP, _F), jnp.float32),
    )(agg1, dinv, W1, b0)


def _tc_final(agg2, dinv, b1, x):
    def body(a_ref, d_ref, b_ref, x_ref, z_ref, z1_ref, z2_ref):
        dinv = d_ref[...]
        xv = x_ref[...]
        z1 = jnp.concatenate([a_ref[0], a_ref[1]], axis=1) * dinv + b_ref[...]
        z2 = jnp.concatenate([a_ref[2], a_ref[3]], axis=1) * dinv + b_ref[...]
        ss = (jnp.sum(z1 * z1, axis=1, keepdims=True)
              + jnp.sum(z2 * z2, axis=1, keepdims=True)
              + jnp.sum(xv * xv, axis=1, keepdims=True))
        rn = lax.rsqrt(ss)
        z1_ref[...] = z1
        z2_ref[...] = z2
        z_ref[:, 0:2 * _F] = z1 * rn
        z_ref[:, 2 * _F:4 * _F] = z2 * rn
        z_ref[:, 4 * _F:] = xv * rn

    g = _N // _R
    return pl.pallas_call(
        body,
        grid=(g,),
        in_specs=[
            pl.BlockSpec((_C, _R, _F), lambda i: (0, i, 0)),
            pl.BlockSpec((_R, 1), lambda i: (i, 0)),
            pl.BlockSpec((1, 2 * _F), lambda i: (0, 0)),
            pl.BlockSpec((_R, _F), lambda i: (i, 0)),
        ],
        out_specs=[
            pl.BlockSpec((_R, 5 * _F), lambda i: (i, 0)),
            pl.BlockSpec((_R, 2 * _F), lambda i: (i, 0)),
            pl.BlockSpec((_R, 2 * _F), lambda i: (i, 0)),
        ],
        out_shape=[
            jax.ShapeDtypeStruct((_N, 5 * _F), jnp.float32),
            jax.ShapeDtypeStruct((_N, 2 * _F), jnp.float32),
            jax.ShapeDtypeStruct((_N, 2 * _F), jnp.float32),
        ],
    )(agg2, dinv, b1, x)


def kernel(edge_index, x_feature, llmfeatures, W0, b0, W1, b1,
           Wp, bp, Wpred, bpred, Wmlp, bmlp):
    src = edge_index[0]
    dst = edge_index[1]
    pad = _EPAD - src.shape[0]
    src_p = jnp.concatenate([src, jnp.zeros((pad,), jnp.int32)])
    dst_p = jnp.concatenate([dst, jnp.full((pad,), _N, jnp.int32)])
    edges3 = jnp.stack([src_p, dst_p]).reshape(2, _EPAD // _EROW, _EROW)

    degp = _sc_degree(edges3)
    degp3 = degp[:, :_N].reshape(2, _N, 1)

    x_cat = jnp.stack([x_feature, llmfeatures])
    hs1, dinv = _tc_first(x_cat, degp3, W0)

    agg1 = _sc_aggregate(hs1.reshape(_C * _NP, _F), edges3)
    hs2 = _tc_mid(agg1.reshape(_C, _NP, _F), dinv, W1, b0.reshape(1, 2 * _F))

    agg2 = _sc_aggregate(hs2.reshape(_C * _NP, _F), edges3)
    z, z1, z2 = _tc_final(agg2.reshape(_C, _NP, _F), dinv,
                          b1.reshape(1, 2 * _F), x_feature)
    return (x_feature, z, z1, z2)


# speedup vs baseline: 7.4726x; 1.0145x over previous
import functools

import jax
import jax.numpy as jnp
from jax import lax
from jax.experimental import pallas as pl
from jax.experimental.pallas import tpu as pltpu
from jax.experimental.pallas import tpu_sc as plsc

_N = 10000
_F = 128
_C = 4
_EROW = 128
_GB = 64
_KB = 8
_NBLK = 20
_NSUB = 16
_EPAD = _NSUB * _NBLK * _KB * _EROW
_NP = 10112
_RPS = _NP // _NSUB
_NPAD = 10240

_HIGH = lax.Precision.HIGHEST


def _sc_degree(edges3):
    mesh = plsc.VectorSubcoreMesh(core_axis_name="c", subcore_axis_name="s", num_cores=2, num_subcores=_NSUB)
    nbd = _EPAD // (32 * _EROW)
    rz = _NPAD // _NSUB

    @functools.partial(
        pl.kernel,
        out_type=jax.ShapeDtypeStruct((2, _NPAD), jnp.float32),
        mesh=mesh,
        scratch_types=[
            pltpu.VMEM((8, _EROW), jnp.int32),
            pltpu.VMEM((_EROW,), jnp.float32),
            pltpu.VMEM((rz,), jnp.float32),
            pltpu.VMEM_SHARED((_NPAD,), jnp.float32),
        ],
    )
    def deg_kernel(e_hbm, o_hbm, dbuf, ones, zbuf, table):
        core = lax.axis_index("c")
        sub = lax.axis_index("s")
        wid = core * _NSUB + sub
        for q in range(_EROW // 16):
            ones[pl.ds(q * 16, 16)] = jnp.ones((16,), jnp.float32)
        for q in range(rz // 16):
            zbuf[pl.ds(q * 16, 16)] = jnp.zeros((16,), jnp.float32)
        pltpu.sync_copy(zbuf, table.at[pl.ds(sub * rz, rz)])
        plsc.subcore_barrier()

        @pl.loop(0, nbd // 8)
        def _(i):
            pltpu.sync_copy(e_hbm.at[1, pl.ds(wid * nbd + i * 8, 8), :], dbuf)
            for k in range(8):
                pltpu.sync_copy(ones, table.at[dbuf.at[k]], add=True)

        plsc.subcore_barrier()
        pltpu.sync_copy(table.at[pl.ds(sub * rz, rz)],
                        o_hbm.at[core, pl.ds(sub * rz, rz)])

    return deg_kernel(edges3)


def _sc_aggregate(hs_flat, edges3):
    mesh = plsc.VectorSubcoreMesh(core_axis_name="c", subcore_axis_name="s", num_cores=2, num_subcores=_NSUB)
    rows_ps = _NBLK * _KB

    @functools.partial(
        pl.kernel,
        out_type=jax.ShapeDtypeStruct((_C * _NP, _F), jnp.float32),
        mesh=mesh,
        scratch_types=[
            pltpu.VMEM((2, 2, _KB, _EROW), jnp.int32),
            pltpu.VMEM((4, _GB), jnp.int32),
            pltpu.VMEM((1, _GB), jnp.int32),
            pltpu.VMEM((4, _GB, _F), jnp.float32),
            pltpu.VMEM_SHARED((_NP, _F), jnp.float32),
            pltpu.SemaphoreType.DMA,
            pltpu.SemaphoreType.DMA,
            pltpu.SemaphoreType.DMA,
            pltpu.SemaphoreType.DMA,
            pltpu.SemaphoreType.DMA,
            pltpu.SemaphoreType.DMA,
        ],
    )
    def agg_kernel(hs_hbm, e_hbm, o_hbm, ibuf, sbuf, dbuf, rbuf, table,
                   si0, si1, sg0, sg1, sg2, sg3):
        core = lax.axis_index("c")
        sub = lax.axis_index("s")
        sem_i = [si0, si1]
        sem_g = [sg0, sg1, sg2, sg3]

        def idx_dma(blk, slot):
            return pltpu.make_async_copy(
                e_hbm.at[:, pl.ds(sub * rows_ps + blk * _KB, _KB), :],
                ibuf.at[slot], sem_i[slot])

        def gather_dma(slot):
            return pltpu.make_async_copy(
                hs_hbm.at[sbuf.at[slot]], rbuf.at[slot], sem_g[slot])

        def comp_srcoff(islot, j, gslot, row0):
            row, off = j // 2, _GB * (j % 2)
            for q in range(_GB // 16):
                sbuf[gslot, pl.ds(q * 16, 16)] = (
                    ibuf[islot, 0, row, pl.ds(off + q * 16, 16)] + row0)

        def scatter_add(islot, j, gslot):
            row, off = j // 2, _GB * (j % 2)
            for q in range(_GB // 16):
                dbuf[0, pl.ds(q * 16, 16)] = (
                    ibuf[islot, 1, row, pl.ds(off + q * 16, 16)])
            pltpu.sync_copy(rbuf.at[gslot], table.at[dbuf.at[0]], add=True)

        nbat = 2 * _KB

        for p in range(2):
            cid = core * 2 + p
            row0 = cid * _NP
            pltpu.sync_copy(hs_hbm.at[pl.ds(row0 + sub * _RPS, _RPS)],
                            table.at[pl.ds(sub * _RPS, _RPS)])
            plsc.subcore_barrier()

            idx_dma(0, 0).start()
            idx_dma(0, 0).wait()
            comp_srcoff(0, 0, 0, row0)
            gather_dma(0).start()
            comp_srcoff(0, 1, 1, row0)
            gather_dma(1).start()
            comp_srcoff(0, 2, 2, row0)
            gather_dma(2).start()

            @pl.loop(0, _NBLK, step=2)
            def _(b):
                for half in range(2):
                    blk = b + half
                    idx_dma(lax.rem(blk + 1, _NBLK), 1 - half).start()
                    for j in range(nbat):
                        cur = j % 4
                        nxt = (j + 3) % 4
                        gather_dma(cur).wait()
                        if j == nbat - 4:
                            idx_dma(0, 1 - half).wait()
                        if j < nbat - 3:
                            comp_srcoff(half, j + 3, nxt, row0)
                        else:
                            comp_srcoff(1 - half, j + 3 - nbat, nxt, row0)
                        gather_dma(nxt).start()
                        scatter_add(half, j, cur)

            gather_dma(0).wait()
            gather_dma(1).wait()
            gather_dma(2).wait()
            plsc.subcore_barrier()
            pltpu.sync_copy(table.at[pl.ds(sub * _RPS, _RPS)],
                            o_hbm.at[pl.ds(row0 + sub * _RPS, _RPS)])
            plsc.subcore_barrier()

    return agg_kernel(hs_flat, edges3)


_R = 1000


def _tc_first(x_cat, degp3, W0):
    def body(x_ref, d_ref, w_ref, o_ref, dinv_ref):
        deg = d_ref[0] + d_ref[1] + 1.0
        dinv = lax.rsqrt(deg)
        dinv_ref[...] = dinv
        for k in range(2):
            h = lax.dot_general(x_ref[k], w_ref[...],
                                (((1,), (0,)), ((), ())), precision=_HIGH)
            hs = h * dinv
            o_ref[2 * k] = hs[:, :_F]
            o_ref[2 * k + 1] = hs[:, _F:]

    g = _N // _R
    return pl.pallas_call(
        body,
        grid=(g,),
        in_specs=[
            pl.BlockSpec((2, _R, _F), lambda i: (0, i, 0)),
            pl.BlockSpec((2, _R, 1), lambda i: (0, i, 0)),
            pl.BlockSpec((_F, 2 * _F), lambda i: (0, 0)),
        ],
        out_specs=[
            pl.BlockSpec((_C, _R, _F), lambda i: (0, i, 0)),
            pl.BlockSpec((_R, 1), lambda i: (i, 0)),
        ],
        out_shape=[
            jax.ShapeDtypeStruct((_C, _NP, _F), jnp.float32),
            jax.ShapeDtypeStruct((_N, 1), jnp.float32),
        ],
    )(x_cat, degp3, W0)


def _tc_mid(agg1, dinv, W1, b0):
    def body(a_ref, d_ref, w_ref, b_ref, o_ref):
        dinv = d_ref[...]
        for k in range(2):
            a = jnp.concatenate([a_ref[2 * k], a_ref[2 * k + 1]], axis=1)
            a = a * dinv + b_ref[...]
            h = lax.dot_general(a, w_ref[...],
                                (((1,), (0,)), ((), ())), precision=_HIGH)
            hs = h * dinv
            o_ref[2 * k] = hs[:, :_F]
            o_ref[2 * k + 1] = hs[:, _F:]

    g = _N // _R
    return pl.pallas_call(
        body,
        grid=(g,),
        in_specs=[
            pl.BlockSpec((_C, _R, _F), lambda i: (0, i, 0)),
            pl.BlockSpec((_R, 1), lambda i: (i, 0)),
            pl.BlockSpec((2 * _F, 2 * _F), lambda i: (0, 0)),
            pl.BlockSpec((1, 2 * _F), lambda i: (0, 0)),
        ],
        out_specs=pl.BlockSpec((_C, _R, _F), lambda i: (0, i, 0)),
        out_shape=jax.ShapeDtypeStruct((_C, _NP, _F), jnp.float32),
    )(agg1, dinv, W1, b0)


def _tc_final(agg2, dinv, b1, x):
    def body(a_ref, d_ref, b_ref, x_ref, z_ref, z1_ref, z2_ref):
        dinv = d_ref[...]
        xv = x_ref[...]
        z1 = jnp.concatenate([a_ref[0], a_ref[1]], axis=1) * dinv + b_ref[...]
        z2 = jnp.concatenate([a_ref[2], a_ref[3]], axis=1) * dinv + b_ref[...]
        ss = (jnp.sum(z1 * z1, axis=1, keepdims=True)
              + jnp.sum(z2 * z2, axis=1, keepdims=True)
              + jnp.sum(xv * xv, axis=1, keepdims=True))
        rn = lax.rsqrt(ss)
        z1_ref[...] = z1
        z2_ref[...] = z2
        z_ref[:, 0:2 * _F] = z1 * rn
        z_ref[:, 2 * _F:4 * _F] = z2 * rn
        z_ref[:, 4 * _F:] = xv * rn

    g = _N // _R
    return pl.pallas_call(
        body,
        grid=(g,),
        in_specs=[
            pl.BlockSpec((_C, _R, _F), lambda i: (0, i, 0)),
            pl.BlockSpec((_R, 1), lambda i: (i, 0)),
            pl.BlockSpec((1, 2 * _F), lambda i: (0, 0)),
            pl.BlockSpec((_R, _F), lambda i: (i, 0)),
        ],
        out_specs=[
            pl.BlockSpec((_R, 5 * _F), lambda i: (i, 0)),
            pl.BlockSpec((_R, 2 * _F), lambda i: (i, 0)),
            pl.BlockSpec((_R, 2 * _F), lambda i: (i, 0)),
        ],
        out_shape=[
            jax.ShapeDtypeStruct((_N, 5 * _F), jnp.float32),
            jax.ShapeDtypeStruct((_N, 2 * _F), jnp.float32),
            jax.ShapeDtypeStruct((_N, 2 * _F), jnp.float32),
        ],
    )(agg2, dinv, b1, x)


def kernel(edge_index, x_feature, llmfeatures, W0, b0, W1, b1,
           Wp, bp, Wpred, bpred, Wmlp, bmlp):
    src = edge_index[0]
    dst = edge_index[1]
    pad = _EPAD - src.shape[0]
    src_p = jnp.concatenate([src, jnp.zeros((pad,), jnp.int32)])
    dst_p = jnp.concatenate([dst, jnp.full((pad,), _N, jnp.int32)])
    edges3 = jnp.stack([src_p, dst_p]).reshape(2, _EPAD // _EROW, _EROW)

    degp = _sc_degree(edges3)
    degp3 = degp[:, :_N].reshape(2, _N, 1)

    x_cat = jnp.stack([x_feature, llmfeatures])
    hs1, dinv = _tc_first(x_cat, degp3, W0)

    agg1 = _sc_aggregate(hs1.reshape(_C * _NP, _F), edges3)
    hs2 = _tc_mid(agg1.reshape(_C, _NP, _F), dinv, W1, b0.reshape(1, 2 * _F))

    agg2 = _sc_aggregate(hs2.reshape(_C * _NP, _F), edges3)
    z, z1, z2 = _tc_final(agg2.reshape(_C, _NP, _F), dinv,
                          b1.reshape(1, 2 * _F), x_feature)
    return (x_feature, z, z1, z2)
